# baseline (speedup 1.0000x reference)
import functools

import jax
import jax.numpy as jnp
from jax import lax
from jax.experimental import pallas as pl
from jax.experimental.pallas import tpu as pltpu
from jax.experimental.pallas import tpu_sc as plsc

D = 64
C = 64
TB = 8192
TBH = TB // 2
SH = 13
MASK = TBH - 1

_BCAST_DNUMS = lax.GatherDimensionNumbers(
    offset_dims=(), collapsed_slice_dims=(0,), start_index_map=(0,))


def _lane_broadcast(vec, i):
    idx = jnp.full((16, 1), i, dtype=jnp.int32)
    return lax.gather(vec, idx, _BCAST_DNUMS, (1,),
                      mode=lax.GatherScatterMode.PROMISE_IN_BOUNDS)


def _repack_body(t_ref, out_ref):
    xt = t_ref[...].T
    out_ref[...] = jnp.concatenate([xt[0:TBH], xt[TBH:]], axis=1)


def _tc_repack(table):
    v, d = table.shape
    grid = (v + TB - 1) // TB
    return pl.pallas_call(
        _repack_body,
        grid=(grid,),
        in_specs=[pl.BlockSpec((d, TB), lambda g: (0, g))],
        out_specs=pl.BlockSpec((TB // 2, 2 * d), lambda g: (g, 0)),
        out_shape=jax.ShapeDtypeStruct((grid * (TB // 2), 2 * d),
                                       jnp.float32),
    )(table.T)


@functools.partial(jax.jit, static_argnames=("n_tokens",))
def _sc_embed(sales_f, item2, text2, consts, tab_i, tab_t, *, n_tokens):
    info = plsc.get_sparse_core_info()
    nc, ns = info.num_cores, info.num_subcores
    nw = nc * ns
    tpw = n_tokens // nw
    nch = tpw // C

    mesh = plsc.VectorSubcoreMesh(core_axis_name="c", subcore_axis_name="s")

    @functools.partial(
        pl.kernel,
        mesh=mesh,
        compiler_params=pltpu.CompilerParams(needs_layout_passes=False),
        out_type=jax.ShapeDtypeStruct((n_tokens, 4 * D), jnp.float32),
        scratch_types=[
            pltpu.VMEM((nch, C), jnp.int32),
            pltpu.VMEM((nch, C), jnp.int32),
            pltpu.VMEM((nch, C), jnp.float32),
            pltpu.VMEM((2, C), jnp.int32),
            pltpu.VMEM((2, C), jnp.int32),
            pltpu.VMEM((2, C, 128), jnp.float32),
            pltpu.VMEM((2, C, 128), jnp.float32),
            pltpu.VMEM((2, C, 4 * D), jnp.float32),
            pltpu.VMEM((3 * D,), jnp.float32),
            pltpu.SemaphoreType.DMA,
            pltpu.SemaphoreType.DMA,
            pltpu.SemaphoreType.DMA,
            pltpu.SemaphoreType.DMA,
        ],
    )
    def body(sales_hbm, item_hbm, text_hbm, consts_hbm, tab_i_hbm, tab_t_hbm,
             out_hbm, idx_i, idx_t, sal, slab_i, slab_t, rows_i, rows_t,
             blocks, cst, gsem0, gsem1, wsem0, wsem1):
        wid = lax.axis_index("s") * nc + lax.axis_index("c")
        base0 = wid * tpw
        gsems = (gsem0, gsem1)
        wsems = (wsem0, wsem1)

        pltpu.sync_copy(consts_hbm, cst)
        pltpu.sync_copy(item_hbm.at[wid], idx_i)
        pltpu.sync_copy(text_hbm.at[wid], idx_t)
        pltpu.sync_copy(sales_hbm.at[wid], sal)

        g = [cst[pl.ds(k * 16, 16)] for k in range(D // 16)]
        w = [cst[pl.ds(D + k * 16, 16)] for k in range(D // 16)]
        b = [cst[pl.ds(2 * D + k * 16, 16)] for k in range(D // 16)]

        def fill_g(t, carry):
            for p in range(2):
                for k in range(D // 16):
                    blocks[p, t, pl.ds(k * 16, 16)] = g[k]
            return carry

        lax.fori_loop(0, C, fill_g, 0)

        def fire(ch, p):
            for k in range(C // 16):
                sl = pl.ds(k * 16, 16)
                ri = idx_i[ch, sl]
                slab_i[p, sl] = ((ri >> SH) << (SH - 1)) | (ri & MASK)
                rt = idx_t[ch, sl]
                slab_t[p, sl] = ((rt >> SH) << (SH - 1)) | (rt & MASK)
            pltpu.async_copy(tab_i_hbm.at[slab_i.at[p]],
                             rows_i.at[p], gsems[p])
            pltpu.async_copy(tab_t_hbm.at[slab_t.at[p]],
                             rows_t.at[p], gsems[p])

        def gwait(p):
            pltpu.make_async_copy(tab_i_hbm.at[slab_i.at[p]],
                                  rows_i.at[p], gsems[p]).wait()
            pltpu.make_async_copy(tab_t_hbm.at[slab_t.at[p]],
                                  rows_t.at[p], gsems[p]).wait()

        fire(0, 0)
        fire(1, 1)

        def duo(gg, carry):
            for p in range(2):
                ch = gg * 2 + p
                base = base0 + ch * C
                gwait(p)

                @pl.when(ch >= 2)
                def _():
                    pltpu.make_async_copy(
                        blocks.at[p], out_hbm.at[pl.ds(base - 2 * C, C)],
                        wsems[p]).wait()

                iota16 = lax.iota(jnp.int32, 16)

                def grp(gi, carry2):
                    t0 = gi * 16
                    sv16 = sal[ch, pl.ds(t0, 16)]
                    for i in range(16):
                        sv = _lane_broadcast(sv16, i)
                        for k in range(D // 16):
                            blocks[p, t0 + i, pl.ds(D + k * 16, 16)] = \
                                sv * w[k] + b[k]
                    tv = t0 + iota16
                    pv = jnp.full((16,), p, dtype=jnp.int32)
                    hoi = ((idx_i[ch, pl.ds(t0, 16)] >> (SH - 1)) & 1) * D
                    hot = ((idx_t[ch, pl.ds(t0, 16)] >> (SH - 1)) & 1) * D
                    for j in range(D):
                        vi = plsc.load_gather(rows_i, [pv, tv, hoi + j])
                        plsc.store_scatter(
                            blocks, [pv, tv, jnp.full((16,), 2 * D + j,
                                                      dtype=jnp.int32)], vi)
                        vt = plsc.load_gather(rows_t, [pv, tv, hot + j])
                        plsc.store_scatter(
                            blocks, [pv, tv, jnp.full((16,), 3 * D + j,
                                                      dtype=jnp.int32)], vt)
                    return carry2

                lax.fori_loop(0, C // 16, grp, 0)

                @pl.when(ch + 2 < nch)
                def _():
                    fire(ch + 2, p)

                pltpu.async_copy(blocks.at[p], out_hbm.at[pl.ds(base, C)],
                                 wsems[p])
            return carry

        lax.fori_loop(0, nch // 2, duo, 0)

        for p in range(2):
            ch = nch - 2 + p
            pltpu.make_async_copy(
                blocks.at[p], out_hbm.at[pl.ds(base0 + ch * C, C)],
                wsems[p]).wait()

    return body(sales_f, item2, text2, consts, tab_i, tab_t)


def kernel(sales, item_id, text, global_token, W_sales, b_sales,
           emb_item, emb_text):
    bsz, seq = item_id.shape
    n = bsz * seq
    nw = 32
    sales_f = (sales.reshape(bsz, seq).T
               .reshape(nw, n // (nw * C), C).astype(jnp.float32))
    item2 = item_id.T.reshape(nw, n // (nw * C), C).astype(jnp.int32)
    text2 = text.T.reshape(nw, n // (nw * C), C).astype(jnp.int32)
    consts = jnp.concatenate([
        global_token.reshape(-1).astype(jnp.float32),
        W_sales.reshape(-1).astype(jnp.float32),
        b_sales.reshape(-1).astype(jnp.float32),
    ])
    tab_i = _tc_repack(emb_item)
    tab_t = _tc_repack(emb_text)
    out = _sc_embed(sales_f, item2, text2, consts, tab_i, tab_t, n_tokens=n)
    return out.reshape(seq, bsz, 4 * D).transpose(1, 0, 2)


# speedup vs baseline: 2.5505x; 2.5505x over previous
import functools

import jax
import jax.numpy as jnp
from jax import lax
from jax.experimental import pallas as pl
from jax.experimental.pallas import tpu as pltpu
from jax.experimental.pallas import tpu_sc as plsc

D = 64
C = 64
TB = 8192

_BCAST_DNUMS = lax.GatherDimensionNumbers(
    offset_dims=(), collapsed_slice_dims=(0,), start_index_map=(0,))


def _lane_broadcast(vec, i):
    idx = jnp.full((16, 1), i, dtype=jnp.int32)
    return lax.gather(vec, idx, _BCAST_DNUMS, (1,),
                      mode=lax.GatherScatterMode.PROMISE_IN_BOUNDS)


def _repack_body(t_ref, out_ref):
    out_ref[:, 0:D] = t_ref[...].T


def _tc_repack(table):
    v, d = table.shape
    grid = (v + TB - 1) // TB
    return pl.pallas_call(
        _repack_body,
        grid=(grid,),
        in_specs=[pl.BlockSpec((d, TB), lambda g: (0, g))],
        out_specs=pl.BlockSpec((TB, 2 * d), lambda g: (g, 0)),
        out_shape=jax.ShapeDtypeStruct((v, 2 * d), jnp.float32),
    )(table.T)


@functools.partial(jax.jit, static_argnames=("n_tokens",))
def _sc_embed(sales_f, item2, text2, consts, tab_i, tab_t, *, n_tokens):
    info = plsc.get_sparse_core_info()
    nc, ns = info.num_cores, info.num_subcores
    nw = nc * ns
    tpw = n_tokens // nw
    nch = tpw // C

    mesh = plsc.VectorSubcoreMesh(core_axis_name="c", subcore_axis_name="s")

    @functools.partial(
        pl.kernel,
        mesh=mesh,
        out_type=jax.ShapeDtypeStruct((n_tokens, 4 * D), jnp.float32),
        scratch_types=[
            pltpu.VMEM((nch, C), jnp.int32),
            pltpu.VMEM((nch, C), jnp.int32),
            pltpu.VMEM((nch, C), jnp.float32),
            pltpu.VMEM((2, C, 128), jnp.float32),
            pltpu.VMEM((2, C, 128), jnp.float32),
            pltpu.VMEM((2, C, 4 * D), jnp.float32),
            pltpu.VMEM((3 * D,), jnp.float32),
            pltpu.SemaphoreType.DMA,
            pltpu.SemaphoreType.DMA,
            pltpu.SemaphoreType.DMA,
            pltpu.SemaphoreType.DMA,
        ],
    )
    def body(sales_hbm, item_hbm, text_hbm, consts_hbm, tab_i_hbm, tab_t_hbm,
             out_hbm, idx_i, idx_t, sal, rows_i, rows_t,
             blocks, cst, gsem0, gsem1, wsem0, wsem1):
        wid = lax.axis_index("s") * nc + lax.axis_index("c")
        base0 = wid * tpw
        gsems = (gsem0, gsem1)
        wsems = (wsem0, wsem1)

        pltpu.sync_copy(consts_hbm, cst)
        pltpu.sync_copy(item_hbm.at[wid], idx_i)
        pltpu.sync_copy(text_hbm.at[wid], idx_t)
        pltpu.sync_copy(sales_hbm.at[wid], sal)

        g = [cst[pl.ds(k * 16, 16)] for k in range(D // 16)]
        w = [cst[pl.ds(D + k * 16, 16)] for k in range(D // 16)]
        b = [cst[pl.ds(2 * D + k * 16, 16)] for k in range(D // 16)]

        def fill_g(t, carry):
            for p in range(2):
                for k in range(D // 16):
                    blocks[p, t, pl.ds(k * 16, 16)] = g[k]
            return carry

        lax.fori_loop(0, C, fill_g, 0)

        def fire(ch, p):
            pltpu.async_copy(tab_i_hbm.at[idx_i.at[ch]],
                             rows_i.at[p], gsems[p])
            pltpu.async_copy(tab_t_hbm.at[idx_t.at[ch]],
                             rows_t.at[p], gsems[p])

        def gwait(ch, p):
            pltpu.make_async_copy(tab_i_hbm.at[idx_i.at[ch]],
                                  rows_i.at[p], gsems[p]).wait()
            pltpu.make_async_copy(tab_t_hbm.at[idx_t.at[ch]],
                                  rows_t.at[p], gsems[p]).wait()

        fire(0, 0)
        fire(1, 1)

        def duo(gg, carry):
            for p in range(2):
                ch = gg * 2 + p
                base = base0 + ch * C
                gwait(ch, p)

                @pl.when(ch >= 2)
                def _():
                    pltpu.make_async_copy(
                        blocks.at[p], out_hbm.at[pl.ds(base - 2 * C, C)],
                        wsems[p]).wait()

                def grp(gi, carry2):
                    t0 = gi * 16
                    sv16 = sal[ch, pl.ds(t0, 16)]
                    for i in range(16):
                        t = t0 + i
                        sv = _lane_broadcast(sv16, i)
                        for k in range(D // 16):
                            s = pl.ds(k * 16, 16)
                            blocks[p, t, pl.ds(D + k * 16, 16)] = \
                                sv * w[k] + b[k]
                            blocks[p, t, pl.ds(2 * D + k * 16, 16)] = \
                                rows_i[p, t, s]
                            blocks[p, t, pl.ds(3 * D + k * 16, 16)] = \
                                rows_t[p, t, s]
                    return carry2

                lax.fori_loop(0, C // 16, grp, 0)

                @pl.when(ch + 2 < nch)
                def _():
                    fire(ch + 2, p)

                pltpu.async_copy(blocks.at[p], out_hbm.at[pl.ds(base, C)],
                                 wsems[p])
            return carry

        lax.fori_loop(0, nch // 2, duo, 0)

        for p in range(2):
            ch = nch - 2 + p
            pltpu.make_async_copy(
                blocks.at[p], out_hbm.at[pl.ds(base0 + ch * C, C)],
                wsems[p]).wait()

    return body(sales_f, item2, text2, consts, tab_i, tab_t)


def kernel(sales, item_id, text, global_token, W_sales, b_sales,
           emb_item, emb_text):
    bsz, seq = item_id.shape
    n = bsz * seq
    nw = 32
    sales_f = (sales.reshape(bsz, seq).T
               .reshape(nw, n // (nw * C), C).astype(jnp.float32))
    item2 = item_id.T.reshape(nw, n // (nw * C), C).astype(jnp.int32)
    text2 = text.T.reshape(nw, n // (nw * C), C).astype(jnp.int32)
    consts = jnp.concatenate([
        global_token.reshape(-1).astype(jnp.float32),
        W_sales.reshape(-1).astype(jnp.float32),
        b_sales.reshape(-1).astype(jnp.float32),
    ])
    tab_i = _tc_repack(emb_item)
    tab_t = _tc_repack(emb_text)
    out = _sc_embed(sales_f, item2, text2, consts, tab_i, tab_t, n_tokens=n)
    return out.reshape(seq, bsz, 4 * D).transpose(1, 0, 2)


# speedup vs baseline: 2.6537x; 1.0404x over previous
import functools

import jax
import jax.numpy as jnp
from jax import lax
from jax.experimental import pallas as pl
from jax.experimental.pallas import tpu as pltpu
from jax.experimental.pallas import tpu_sc as plsc

D = 64
C = 64
TB = 16384

_BCAST_DNUMS = lax.GatherDimensionNumbers(
    offset_dims=(), collapsed_slice_dims=(0,), start_index_map=(0,))


def _lane_broadcast(vec, i):
    idx = jnp.full((16, 1), i, dtype=jnp.int32)
    return lax.gather(vec, idx, _BCAST_DNUMS, (1,),
                      mode=lax.GatherScatterMode.PROMISE_IN_BOUNDS)


def _repack_body(t_ref, out_ref):
    out_ref[:, 0:D] = t_ref[...].T


def _tc_repack(table):
    v, d = table.shape
    grid = (v + TB - 1) // TB
    return pl.pallas_call(
        _repack_body,
        grid=(grid,),
        in_specs=[pl.BlockSpec((d, TB), lambda g: (0, g))],
        out_specs=pl.BlockSpec((TB, 2 * d), lambda g: (g, 0)),
        out_shape=jax.ShapeDtypeStruct((v, 2 * d), jnp.float32),
    )(table.T)


@functools.partial(jax.jit, static_argnames=("n_tokens",))
def _sc_embed(sales_f, item2, text2, consts, tab_i, tab_t, *, n_tokens):
    info = plsc.get_sparse_core_info()
    nc, ns = info.num_cores, info.num_subcores
    nw = nc * ns
    tpw = n_tokens // nw
    nch = tpw // C

    mesh = plsc.VectorSubcoreMesh(core_axis_name="c", subcore_axis_name="s")

    @functools.partial(
        pl.kernel,
        mesh=mesh,
        out_type=jax.ShapeDtypeStruct((n_tokens, 4 * D), jnp.float32),
        scratch_types=[
            pltpu.VMEM((nch, C), jnp.int32),
            pltpu.VMEM((nch, C), jnp.int32),
            pltpu.VMEM((nch, C), jnp.float32),
            pltpu.VMEM((2, C, 128), jnp.float32),
            pltpu.VMEM((2, C, 128), jnp.float32),
            pltpu.VMEM((2, C, 4 * D), jnp.float32),
            pltpu.VMEM((3 * D,), jnp.float32),
            pltpu.SemaphoreType.DMA,
            pltpu.SemaphoreType.DMA,
            pltpu.SemaphoreType.DMA,
            pltpu.SemaphoreType.DMA,
        ],
    )
    def body(sales_hbm, item_hbm, text_hbm, consts_hbm, tab_i_hbm, tab_t_hbm,
             out_hbm, idx_i, idx_t, sal, rows_i, rows_t,
             blocks, cst, gsem0, gsem1, wsem0, wsem1):
        wid = lax.axis_index("s") * nc + lax.axis_index("c")
        base0 = wid * tpw
        gsems = (gsem0, gsem1)
        wsems = (wsem0, wsem1)

        pltpu.sync_copy(consts_hbm, cst)
        pltpu.sync_copy(item_hbm.at[wid], idx_i)
        pltpu.sync_copy(text_hbm.at[wid], idx_t)
        pltpu.sync_copy(sales_hbm.at[wid], sal)

        g = [cst[pl.ds(k * 16, 16)] for k in range(D // 16)]
        w = [cst[pl.ds(D + k * 16, 16)] for k in range(D // 16)]
        b = [cst[pl.ds(2 * D + k * 16, 16)] for k in range(D // 16)]

        def fill_g(t, carry):
            for p in range(2):
                for k in range(D // 16):
                    blocks[p, t, pl.ds(k * 16, 16)] = g[k]
            return carry

        lax.fori_loop(0, C, fill_g, 0)

        def fire(ch, p):
            pltpu.async_copy(tab_i_hbm.at[idx_i.at[ch]],
                             rows_i.at[p], gsems[p])
            pltpu.async_copy(tab_t_hbm.at[idx_t.at[ch]],
                             rows_t.at[p], gsems[p])

        def gwait(ch, p):
            pltpu.make_async_copy(tab_i_hbm.at[idx_i.at[ch]],
                                  rows_i.at[p], gsems[p]).wait()
            pltpu.make_async_copy(tab_t_hbm.at[idx_t.at[ch]],
                                  rows_t.at[p], gsems[p]).wait()

        fire(0, 0)
        fire(1, 1)

        def duo(gg, carry):
            for p in range(2):
                ch = gg * 2 + p
                base = base0 + ch * C
                gwait(ch, p)

                @pl.when(ch >= 2)
                def _():
                    pltpu.make_async_copy(
                        blocks.at[p], out_hbm.at[pl.ds(base - 2 * C, C)],
                        wsems[p]).wait()

                def grp(gi, carry2):
                    t0 = gi * 16
                    sv16 = sal[ch, pl.ds(t0, 16)]
                    for i in range(16):
                        t = t0 + i
                        sv = _lane_broadcast(sv16, i)
                        for k in range(D // 16):
                            s = pl.ds(k * 16, 16)
                            blocks[p, t, pl.ds(D + k * 16, 16)] = \
                                sv * w[k] + b[k]
                            blocks[p, t, pl.ds(2 * D + k * 16, 16)] = \
                                rows_i[p, t, s]
                            blocks[p, t, pl.ds(3 * D + k * 16, 16)] = \
                                rows_t[p, t, s]
                    return carry2

                lax.fori_loop(0, C // 16, grp, 0)

                @pl.when(ch + 2 < nch)
                def _():
                    fire(ch + 2, p)

                pltpu.async_copy(blocks.at[p], out_hbm.at[pl.ds(base, C)],
                                 wsems[p])
            return carry

        lax.fori_loop(0, nch // 2, duo, 0)

        for p in range(2):
            ch = nch - 2 + p
            pltpu.make_async_copy(
                blocks.at[p], out_hbm.at[pl.ds(base0 + ch * C, C)],
                wsems[p]).wait()

    return body(sales_f, item2, text2, consts, tab_i, tab_t)


def kernel(sales, item_id, text, global_token, W_sales, b_sales,
           emb_item, emb_text):
    bsz, seq = item_id.shape
    n = bsz * seq
    nw = 32
    sales_f = (sales.reshape(bsz, seq).T
               .reshape(nw, n // (nw * C), C).astype(jnp.float32))
    item2 = item_id.T.reshape(nw, n // (nw * C), C).astype(jnp.int32)
    text2 = text.T.reshape(nw, n // (nw * C), C).astype(jnp.int32)
    consts = jnp.concatenate([
        global_token.reshape(-1).astype(jnp.float32),
        W_sales.reshape(-1).astype(jnp.float32),
        b_sales.reshape(-1).astype(jnp.float32),
    ])
    tab_i = _tc_repack(emb_item)
    tab_t = _tc_repack(emb_text)
    out = _sc_embed(sales_f, item2, text2, consts, tab_i, tab_t, n_tokens=n)
    return out.reshape(seq, bsz, 4 * D).transpose(1, 0, 2)


# speedup vs baseline: 2.6569x; 1.0012x over previous
import functools

import jax
import jax.numpy as jnp
from jax import lax
from jax.experimental import pallas as pl
from jax.experimental.pallas import tpu as pltpu
from jax.experimental.pallas import tpu_sc as plsc

D = 64
C = 80
TB = 16384

_BCAST_DNUMS = lax.GatherDimensionNumbers(
    offset_dims=(), collapsed_slice_dims=(0,), start_index_map=(0,))


def _lane_broadcast(vec, i):
    idx = jnp.full((16, 1), i, dtype=jnp.int32)
    return lax.gather(vec, idx, _BCAST_DNUMS, (1,),
                      mode=lax.GatherScatterMode.PROMISE_IN_BOUNDS)


def _repack_body(t_ref, out_ref):
    out_ref[:, 0:D] = t_ref[...].T


def _tc_repack(table):
    v, d = table.shape
    grid = (v + TB - 1) // TB
    return pl.pallas_call(
        _repack_body,
        grid=(grid,),
        in_specs=[pl.BlockSpec((d, TB), lambda g: (0, g))],
        out_specs=pl.BlockSpec((TB, 2 * d), lambda g: (g, 0)),
        out_shape=jax.ShapeDtypeStruct((v, 2 * d), jnp.float32),
    )(table.T)


@functools.partial(jax.jit, static_argnames=("n_tokens",))
def _sc_embed(sales_f, item2, text2, consts, tab_i, tab_t, *, n_tokens):
    info = plsc.get_sparse_core_info()
    nc, ns = info.num_cores, info.num_subcores
    nw = nc * ns
    tpw = n_tokens // nw
    nch = tpw // C

    mesh = plsc.VectorSubcoreMesh(core_axis_name="c", subcore_axis_name="s")

    @functools.partial(
        pl.kernel,
        mesh=mesh,
        out_type=jax.ShapeDtypeStruct((n_tokens, 4 * D), jnp.float32),
        scratch_types=[
            pltpu.VMEM((nch, C), jnp.int32),
            pltpu.VMEM((nch, C), jnp.int32),
            pltpu.VMEM((nch, C), jnp.float32),
            pltpu.VMEM((2, C, 128), jnp.float32),
            pltpu.VMEM((2, C, 128), jnp.float32),
            pltpu.VMEM((2, C, 4 * D), jnp.float32),
            pltpu.VMEM((3 * D,), jnp.float32),
            pltpu.SemaphoreType.DMA,
            pltpu.SemaphoreType.DMA,
            pltpu.SemaphoreType.DMA,
            pltpu.SemaphoreType.DMA,
        ],
    )
    def body(sales_hbm, item_hbm, text_hbm, consts_hbm, tab_i_hbm, tab_t_hbm,
             out_hbm, idx_i, idx_t, sal, rows_i, rows_t,
             blocks, cst, gsem0, gsem1, wsem0, wsem1):
        wid = lax.axis_index("s") * nc + lax.axis_index("c")
        base0 = wid * tpw
        gsems = (gsem0, gsem1)
        wsems = (wsem0, wsem1)

        pltpu.sync_copy(consts_hbm, cst)
        pltpu.sync_copy(item_hbm.at[wid], idx_i)
        pltpu.sync_copy(text_hbm.at[wid], idx_t)
        pltpu.sync_copy(sales_hbm.at[wid], sal)

        g = [cst[pl.ds(k * 16, 16)] for k in range(D // 16)]
        w = [cst[pl.ds(D + k * 16, 16)] for k in range(D // 16)]
        b = [cst[pl.ds(2 * D + k * 16, 16)] for k in range(D // 16)]

        def fill_g(t, carry):
            for p in range(2):
                for k in range(D // 16):
                    blocks[p, t, pl.ds(k * 16, 16)] = g[k]
            return carry

        lax.fori_loop(0, C, fill_g, 0)

        def fire(ch, p):
            pltpu.async_copy(tab_i_hbm.at[idx_i.at[ch]],
                             rows_i.at[p], gsems[p])
            pltpu.async_copy(tab_t_hbm.at[idx_t.at[ch]],
                             rows_t.at[p], gsems[p])

        def gwait(ch, p):
            pltpu.make_async_copy(tab_i_hbm.at[idx_i.at[ch]],
                                  rows_i.at[p], gsems[p]).wait()
            pltpu.make_async_copy(tab_t_hbm.at[idx_t.at[ch]],
                                  rows_t.at[p], gsems[p]).wait()

        fire(0, 0)
        fire(1, 1)

        def duo(gg, carry):
            for p in range(2):
                ch = gg * 2 + p
                base = base0 + ch * C
                gwait(ch, p)

                @pl.when(ch >= 2)
                def _():
                    pltpu.make_async_copy(
                        blocks.at[p], out_hbm.at[pl.ds(base - 2 * C, C)],
                        wsems[p]).wait()

                def grp(gi, carry2):
                    t0 = gi * 16
                    sv16 = sal[ch, pl.ds(t0, 16)]
                    for i in range(16):
                        t = t0 + i
                        sv = _lane_broadcast(sv16, i)
                        for k in range(D // 16):
                            s = pl.ds(k * 16, 16)
                            blocks[p, t, pl.ds(D + k * 16, 16)] = \
                                sv * w[k] + b[k]
                            blocks[p, t, pl.ds(2 * D + k * 16, 16)] = \
                                rows_i[p, t, s]
                            blocks[p, t, pl.ds(3 * D + k * 16, 16)] = \
                                rows_t[p, t, s]
                    return carry2

                lax.fori_loop(0, C // 16, grp, 0)

                @pl.when(ch + 2 < nch)
                def _():
                    fire(ch + 2, p)

                pltpu.async_copy(blocks.at[p], out_hbm.at[pl.ds(base, C)],
                                 wsems[p])
            return carry

        lax.fori_loop(0, nch // 2, duo, 0)

        for p in range(2):
            ch = nch - 2 + p
            pltpu.make_async_copy(
                blocks.at[p], out_hbm.at[pl.ds(base0 + ch * C, C)],
                wsems[p]).wait()

    return body(sales_f, item2, text2, consts, tab_i, tab_t)


def kernel(sales, item_id, text, global_token, W_sales, b_sales,
           emb_item, emb_text):
    bsz, seq = item_id.shape
    n = bsz * seq
    nw = 32
    sales_f = (sales.reshape(bsz, seq).T
               .reshape(nw, n // (nw * C), C).astype(jnp.float32))
    item2 = item_id.T.reshape(nw, n // (nw * C), C).astype(jnp.int32)
    text2 = text.T.reshape(nw, n // (nw * C), C).astype(jnp.int32)
    consts = jnp.concatenate([
        global_token.reshape(-1).astype(jnp.float32),
        W_sales.reshape(-1).astype(jnp.float32),
        b_sales.reshape(-1).astype(jnp.float32),
    ])
    tab_i = _tc_repack(emb_item)
    tab_t = _tc_repack(emb_text)
    out = _sc_embed(sales_f, item2, text2, consts, tab_i, tab_t, n_tokens=n)
    return out.reshape(seq, bsz, 4 * D).transpose(1, 0, 2)
